# baseline (speedup 1.0000x reference)
import functools

import jax
import jax.numpy as jnp
from jax import lax
from jax.experimental import pallas as pl
from jax.experimental.pallas import tpu as pltpu
from jax.experimental.pallas import tpu_sc as plsc

_NC = 2
_NS = 16
_NW = _NC * _NS
_L = 16


@functools.partial(jax.jit, static_argnames=("batch", "seq", "d_model"))
def _gather_sc(idx_flat, tpairs, *, batch, seq, d_model):
    b_per_w = batch // _NW
    n_per_w = b_per_w * seq
    n_chunks = b_per_w
    assert n_chunks % 2 == 0 and n_per_w % _L == 0
    nq = d_model // _L

    mesh = plsc.VectorSubcoreMesh(core_axis_name="c", subcore_axis_name="s")

    @functools.partial(
        pl.kernel,
        out_type=jax.ShapeDtypeStruct((batch, seq, d_model), jnp.float32),
        mesh=mesh,
        scratch_types=[
            pltpu.VMEM((n_per_w,), jnp.int32),
            [pltpu.VMEM((seq,), jnp.int32) for _ in range(2)],
            [pltpu.VMEM((seq, 2 * d_model), jnp.float32) for _ in range(2)],
            [pltpu.VMEM((seq, d_model), jnp.float32) for _ in range(2)],
            [pltpu.SemaphoreType.DMA for _ in range(2)],
            [pltpu.SemaphoreType.DMA for _ in range(2)],
        ],
        compiler_params=pltpu.CompilerParams(use_tc_tiling_on_sc=True, needs_layout_passes=False),
    )
    def k(idx_hbm, tp_hbm, out_hbm, pidx_v, raws, pbufs, obufs, g_sems, o_sems):
        wid = lax.axis_index("s") * _NC + lax.axis_index("c")
        base = wid * n_per_w
        b0 = wid * b_per_w

        pltpu.sync_copy(idx_hbm.at[pl.ds(base, n_per_w)], pidx_v)

        @pl.loop(0, n_per_w, step=_L)
        def _xform(i):
            pidx_v[pl.ds(i, _L)] = jax.lax.shift_right_logical(
                pidx_v[pl.ds(i, _L)], 1
            )

        def fire_gather(j, s):
            pltpu.async_copy(
                tp_hbm.at[pidx_v.at[pl.ds(j * seq, seq)]], pbufs[s], g_sems[s]
            )

        def wait_gather(s):
            pltpu.make_async_copy(
                tp_hbm.at[pidx_v.at[pl.ds(0, seq)]], pbufs[s], g_sems[s]
            ).wait()

        def fire_put(j, s):
            pltpu.async_copy(obufs[s], out_hbm.at[b0 + j], o_sems[s])

        def wait_put(s):
            pltpu.make_async_copy(obufs[s], out_hbm.at[b0], o_sems[s]).wait()

        def select(j, s):
            pbuf, obuf = pbufs[s], obufs[s]
            pltpu.sync_copy(idx_hbm.at[pl.ds(base + j * seq, seq)], raws[s])

            @pl.loop(0, seq)
            def _rows(r):
                h16 = plsc.load_gather(
                    raws[s], [jnp.broadcast_to(r, (_L,))]
                )
                take_hi = jax.lax.bitwise_and(h16, 1) != 0
                for q in range(nq):
                    a = pbuf[r, pl.ds(q * _L, _L)]
                    b = pbuf[r, pl.ds(d_model + q * _L, _L)]
                    obuf[r, pl.ds(q * _L, _L)] = jnp.where(take_hi, b, a)

        fire_gather(0, 0)

        @pl.loop(0, n_chunks - 2, step=2)
        def _body(j0):
            not_first = j0 != 0

            @pl.when(not_first)
            def _():
                wait_put(1)

            fire_gather(j0 + 1, 1)
            wait_gather(0)

            @pl.when(not_first)
            def _():
                wait_put(0)

            select(j0, 0)
            fire_put(j0, 0)
            fire_gather(j0 + 2, 0)
            wait_gather(1)
            select(j0 + 1, 1)
            fire_put(j0 + 1, 1)

        jn = n_chunks - 2
        wait_put(1)
        fire_gather(jn + 1, 1)
        wait_gather(0)
        wait_put(0)
        select(jn, 0)
        fire_put(jn, 0)
        wait_gather(1)
        select(jn + 1, 1)
        fire_put(jn + 1, 1)
        wait_put(0)
        wait_put(1)

    return k(idx_flat, tpairs)


def kernel(input_ids, table):
    batch, seq = input_ids.shape
    vocab, d_model = table.shape
    idx_flat = input_ids.reshape(batch * seq).astype(jnp.int32)
    tpairs = table.reshape(vocab // 2, 2 * d_model)
    return _gather_sc(idx_flat, tpairs, batch=batch, seq=seq, d_model=d_model)


# speedup vs baseline: 1.2614x; 1.2614x over previous
import functools

import jax
import jax.numpy as jnp
from jax import lax
from jax.experimental import pallas as pl
from jax.experimental.pallas import tpu as pltpu
from jax.experimental.pallas import tpu_sc as plsc

_NC = 2
_NS = 16
_NW = _NC * _NS

_NB = 4


@functools.partial(jax.jit, static_argnames=("batch", "seq", "d_model"))
def _gather_sc(idx_flat, table, *, batch, seq, d_model):
    b_per_w = batch // _NW
    n_per_w = b_per_w * seq
    n_chunks = b_per_w
    assert n_chunks % (2 * _NB) == 0 and n_chunks >= 4 * _NB

    mesh = plsc.VectorSubcoreMesh(core_axis_name="c", subcore_axis_name="s")

    @functools.partial(
        pl.kernel,
        out_type=jax.ShapeDtypeStruct((batch, seq, d_model), jnp.float32),
        mesh=mesh,
        scratch_types=[
            pltpu.VMEM((n_per_w,), jnp.int32),
            [pltpu.VMEM((seq, d_model), jnp.float32) for _ in range(2 * _NB)],
            [pltpu.SemaphoreType.DMA for _ in range(2 * _NB)],
            [pltpu.SemaphoreType.DMA for _ in range(2 * _NB)],
        ],
        compiler_params=pltpu.CompilerParams(use_tc_tiling_on_sc=False),
    )
    def k(idx_hbm, table_hbm, out_hbm, idx_v, bufs, g_sems, o_sems):
        wid = lax.axis_index("s") * _NC + lax.axis_index("c")
        base = wid * n_per_w
        b0 = wid * b_per_w

        pltpu.sync_copy(idx_hbm.at[pl.ds(base, n_per_w)], idx_v)

        def fire_gather(j, s):
            pltpu.async_copy(
                table_hbm.at[idx_v.at[pl.ds(j * seq, seq)]],
                bufs[s],
                g_sems[s],
            )

        def wait_gather(s):
            pltpu.make_async_copy(
                table_hbm.at[idx_v.at[pl.ds(0, seq)]], bufs[s], g_sems[s]
            ).wait()

        def fire_put(j, s):
            pltpu.async_copy(bufs[s], out_hbm.at[b0 + j], o_sems[s])

        def wait_put(s):
            pltpu.make_async_copy(bufs[s], out_hbm.at[b0], o_sems[s]).wait()

        def half(j0, grp, fire, do_owait):
            for b in range(_NB):
                j = j0 + b
                s = grp * _NB + b
                ns = (1 - grp) * _NB + b
                if fire:
                    if do_owait is None:
                        wait_put(ns)
                    else:
                        @pl.when(do_owait)
                        def _():
                            wait_put(ns)

                    fire_gather(j + _NB, ns)
                wait_gather(s)
                fire_put(j, s)

        for b in range(_NB):
            fire_gather(b, b)

        @pl.loop(0, n_chunks - 2 * _NB, step=2 * _NB)
        def _body(j0):
            half(j0, 0, True, j0 != 0)
            half(j0 + _NB, 1, True, None)

        j0 = n_chunks - 2 * _NB
        half(j0, 0, True, None)
        half(j0 + _NB, 1, False, None)
        for s in range(2 * _NB):
            wait_put(s)

    return k(idx_flat, table)


def kernel(input_ids, table):
    batch, seq = input_ids.shape
    vocab, d_model = table.shape
    idx_flat = input_ids.reshape(batch * seq).astype(jnp.int32)
    return _gather_sc(idx_flat, table, batch=batch, seq=seq, d_model=d_model)
